# baseline (speedup 1.0000x reference)
import jax
import jax.numpy as jnp
from jax import lax
from jax.experimental import pallas as pl
from jax.experimental.pallas import tpu as pltpu
from jax.experimental.pallas import tpu_sc as plsc

_T = 2048
_D = 1024
_E = 64
_K = 2
_FF = 512
_SFF = 512
_P = _T * _K
_TILE = 128
_NT_PAD = _P + _E * _TILE
_NUM_TILES = _NT_PAD // _TILE
_TT = 128

_NC = 2
_NS = 16
_NW = _NC * _NS
_BPW = _P // _NW


def _cumsum_rows(x):
    n = x.shape[0]
    sh = 1
    while sh < n:
        pad = jnp.zeros((sh, x.shape[1]), x.dtype)
        x = x + jnp.concatenate([pad, x[:-sh]], axis=0)
        sh *= 2
    return x


def _cumsum_lanes(x):
    n = x.shape[1]
    sh = 1
    while sh < n:
        pad = jnp.zeros((x.shape[0], sh), x.dtype)
        x = x + jnp.concatenate([pad, x[:, :-sh]], axis=1)
        sh *= 2
    return x


def _router_kernel(x_ref, gw_ref, w_ref, pos_ref, te_ref):
    x = x_ref[...]
    gw = gw_ref[...]
    logits = lax.dot_general(
        x, gw, (((1,), (1,)), ((), ())),
        preferred_element_type=jnp.float32)

    l1 = jnp.max(logits, axis=1, keepdims=True)
    i1 = jnp.argmax(logits, axis=1, keepdims=True)
    ecol = lax.broadcasted_iota(jnp.int32, (_T, _E), 1)
    masked = jnp.where(ecol == i1, -jnp.inf, logits)
    l2 = jnp.max(masked, axis=1, keepdims=True)
    i2 = jnp.argmax(masked, axis=1, keepdims=True)
    r = jnp.exp(l2 - l1)
    s = 1.0 + r
    w_ref[...] = jnp.concatenate([1.0 / s, r / s], axis=1)

    oh = jnp.concatenate([(ecol == i1), (ecol == i2)],
                         axis=0).astype(jnp.int32)
    csum = _cumsum_rows(oh)
    counts = csum[_P - 1:_P, :]
    rank = jnp.sum(oh * csum, axis=1, keepdims=True) - 1
    pc = ((counts + (_TILE - 1)) // _TILE) * _TILE
    cpc = _cumsum_lanes(pc)
    po = cpc - pc
    pos_flat = jnp.sum(oh * po, axis=1, keepdims=True) + rank
    pos_ref[...] = jnp.concatenate([pos_flat[:_T], pos_flat[_T:]], axis=1)

    trow = lax.broadcasted_iota(jnp.int32, (_NUM_TILES, _E), 0) * _TILE
    te = jnp.sum((trow >= cpc).astype(jnp.int32), axis=1, keepdims=True)
    te_ref[...] = jnp.minimum(te, _E - 1)


def _sc_scatter_x(x_i32, pos_row):
    mesh = plsc.VectorSubcoreMesh(core_axis_name="c", subcore_axis_name="s")

    @pl.kernel(
        mesh=mesh,
        out_type=jax.ShapeDtypeStruct((_NT_PAD, _D // 2), jnp.int32),
        scratch_types=[
            pltpu.VMEM((_BPW,), jnp.int32),
            pltpu.VMEM((_BPW, _D // 2), jnp.int32),
            pltpu.SemaphoreType.DMA,
        ],
    )
    def k(x_hbm, idx_hbm, out_hbm, idx_v, rows_v, sem):
        wid = lax.axis_index("s") * _NC + lax.axis_index("c")
        base = wid * _BPW
        src = lax.rem(base, _T)
        pltpu.sync_copy(idx_hbm.at[pl.ds(base, _BPW)], idx_v)
        pltpu.sync_copy(x_hbm.at[pl.ds(src, _BPW)], rows_v)
        pltpu.async_copy(rows_v, out_hbm.at[idx_v], sem).wait()

    return k(x_i32, pos_row)


def _sc_gather_os(os_i32, pos_row):
    mesh = plsc.VectorSubcoreMesh(core_axis_name="c", subcore_axis_name="s")

    @pl.kernel(
        mesh=mesh,
        out_type=jax.ShapeDtypeStruct((_P, _D // 2), jnp.int32),
        scratch_types=[
            pltpu.VMEM((_BPW,), jnp.int32),
            pltpu.VMEM((_BPW, _D // 2), jnp.int32),
            pltpu.SemaphoreType.DMA,
        ],
    )
    def k(os_hbm, idx_hbm, out_hbm, idx_v, rows_v, sem):
        wid = lax.axis_index("s") * _NC + lax.axis_index("c")
        base = wid * _BPW
        pltpu.sync_copy(idx_hbm.at[pl.ds(base, _BPW)], idx_v)
        pltpu.async_copy(os_hbm.at[idx_v], rows_v, sem).wait()
        pltpu.sync_copy(rows_v, out_hbm.at[pl.ds(base, _BPW)])

    return k(os_i32, pos_row)


def _expert_kernel(te_ref, xs_ref, w1_ref, w2_ref, out_ref):
    del te_ref
    xg = xs_ref[...]
    h = jnp.dot(xg, w1_ref[0].astype(jnp.bfloat16),
                preferred_element_type=jnp.float32)
    g = h[:, :_FF]
    u = h[:, _FF:]
    a = (jax.nn.silu(g) * u).astype(jnp.bfloat16)
    o = jnp.dot(a, w2_ref[0].astype(jnp.bfloat16),
                preferred_element_type=jnp.float32)
    out_ref[...] = o.astype(jnp.bfloat16)


def _shared_kernel(x_ref, sgu_ref, sd_ref, out_ref):
    h = jnp.dot(x_ref[...], sgu_ref[...], preferred_element_type=jnp.float32)
    g = h[:, :_SFF]
    u = h[:, _SFF:]
    a = (jax.nn.silu(g) * u).astype(jnp.bfloat16)
    out_ref[...] = jnp.dot(a, sd_ref[...], preferred_element_type=jnp.float32)


def _final_kernel(sh_ref, g0_ref, g1_ref, w_ref, out_ref):
    w0 = w_ref[:, 0:1]
    w1 = w_ref[:, 1:2]
    out_ref[...] = (sh_ref[...]
                    + w0 * g0_ref[...].astype(jnp.float32)
                    + w1 * g1_ref[...].astype(jnp.float32))


def kernel(hidden_states, gate_w, w_gate_up, w_down, shared_gate_up, shared_down):
    x_bf = hidden_states.astype(jnp.bfloat16)
    sgu_bf = shared_gate_up.astype(jnp.bfloat16)
    sd_bf = shared_down.astype(jnp.bfloat16)

    topk_w, pos2, te = pl.pallas_call(
        _router_kernel,
        out_shape=[
            jax.ShapeDtypeStruct((_T, _K), jnp.float32),
            jax.ShapeDtypeStruct((_T, _K), jnp.int32),
            jax.ShapeDtypeStruct((_NUM_TILES, 1), jnp.int32),
        ],
    )(hidden_states, gate_w)

    te_flat = te.reshape(_NUM_TILES)
    pos_row = pos2.T.reshape(_P)

    x_i32 = lax.bitcast_convert_type(
        x_bf.reshape(_T, _D // 2, 2), jnp.int32)
    xs_i32 = _sc_scatter_x(x_i32, pos_row)
    x_sorted = lax.bitcast_convert_type(
        xs_i32, jnp.bfloat16).reshape(_NT_PAD, _D)

    os = pl.pallas_call(
        _expert_kernel,
        grid_spec=pltpu.PrefetchScalarGridSpec(
            num_scalar_prefetch=1,
            grid=(_NUM_TILES,),
            in_specs=[
                pl.BlockSpec((_TILE, _D), lambda t, te: (t, 0)),
                pl.BlockSpec((1, _D, 2 * _FF), lambda t, te: (te[t], 0, 0)),
                pl.BlockSpec((1, _FF, _D), lambda t, te: (te[t], 0, 0)),
            ],
            out_specs=pl.BlockSpec((_TILE, _D), lambda t, te: (t, 0)),
        ),
        out_shape=jax.ShapeDtypeStruct((_NT_PAD, _D), jnp.bfloat16),
    )(te_flat, x_sorted, w_gate_up, w_down)

    shared = pl.pallas_call(
        _shared_kernel,
        grid=(_T // _TT,),
        in_specs=[
            pl.BlockSpec((_TT, _D), lambda i: (i, 0)),
            pl.BlockSpec((_D, 2 * _SFF), lambda i: (0, 0)),
            pl.BlockSpec((_SFF, _D), lambda i: (0, 0)),
        ],
        out_specs=pl.BlockSpec((_TT, _D), lambda i: (i, 0)),
        out_shape=jax.ShapeDtypeStruct((_T, _D), jnp.float32),
    )(x_bf, sgu_bf, sd_bf)

    os_i32 = lax.bitcast_convert_type(
        os.reshape(_NT_PAD, _D // 2, 2), jnp.int32)
    g_i32 = _sc_gather_os(os_i32, pos_row)
    g = lax.bitcast_convert_type(g_i32, jnp.bfloat16).reshape(_P, _D)

    out = pl.pallas_call(
        _final_kernel,
        grid=(_T // _TT,),
        in_specs=[
            pl.BlockSpec((_TT, _D), lambda i: (i, 0)),
            pl.BlockSpec((_TT, _D), lambda i: (i, 0)),
            pl.BlockSpec((_TT, _D), lambda i: (i + _T // _TT, 0)),
            pl.BlockSpec((_TT, _K), lambda i: (i, 0)),
        ],
        out_specs=pl.BlockSpec((_TT, _D), lambda i: (i, 0)),
        out_shape=jax.ShapeDtypeStruct((_T, _D), jnp.float32),
    )(shared, g, g, topk_w)

    return out


# speedup vs baseline: 3.5232x; 3.5232x over previous
import jax
import jax.numpy as jnp
from jax import lax
from jax.experimental import pallas as pl
from jax.experimental.pallas import tpu as pltpu
from jax.experimental.pallas import tpu_sc as plsc

_T = 2048
_D = 1024
_E = 64
_K = 2
_FF = 512
_SFF = 512
_P = _T * _K
_TILE = 128
_NT_PAD = _P + _E * _TILE
_NUM_TILES = _NT_PAD // _TILE
_TT = 128

_NC = 2
_NS = 16
_NW = _NC * _NS
_BPW = _P // _NW


def _cumsum_rows(x):
    n = x.shape[0]
    sh = 1
    while sh < n:
        pad = jnp.zeros((sh, x.shape[1]), x.dtype)
        x = x + jnp.concatenate([pad, x[:-sh]], axis=0)
        sh *= 2
    return x


def _cumsum_lanes(x):
    n = x.shape[1]
    sh = 1
    while sh < n:
        pad = jnp.zeros((x.shape[0], sh), x.dtype)
        x = x + jnp.concatenate([pad, x[:, :-sh]], axis=1)
        sh *= 2
    return x


def _router_kernel(x_ref, gw_ref, w_ref, pos_ref, te_ref):
    x = x_ref[...]
    gw = gw_ref[...]
    logits = lax.dot_general(
        x, gw, (((1,), (1,)), ((), ())),
        preferred_element_type=jnp.float32)

    l1 = jnp.max(logits, axis=1, keepdims=True)
    i1 = jnp.argmax(logits, axis=1, keepdims=True)
    ecol = lax.broadcasted_iota(jnp.int32, (_T, _E), 1)
    masked = jnp.where(ecol == i1, -jnp.inf, logits)
    l2 = jnp.max(masked, axis=1, keepdims=True)
    i2 = jnp.argmax(masked, axis=1, keepdims=True)
    r = jnp.exp(l2 - l1)
    s = 1.0 + r
    w_ref[...] = jnp.concatenate([1.0 / s, r / s], axis=1)

    oh = jnp.concatenate([(ecol == i1), (ecol == i2)],
                         axis=0).astype(jnp.int32)
    csum = _cumsum_rows(oh)
    counts = csum[_P - 1:_P, :]
    rank = jnp.sum(oh * csum, axis=1, keepdims=True) - 1
    pc = ((counts + (_TILE - 1)) // _TILE) * _TILE
    cpc = _cumsum_lanes(pc)
    po = cpc - pc
    pos_flat = jnp.sum(oh * po, axis=1, keepdims=True) + rank
    pos_ref[...] = jnp.concatenate([pos_flat[:_T], pos_flat[_T:]], axis=1)

    trow = lax.broadcasted_iota(jnp.int32, (_NUM_TILES, _E), 0) * _TILE
    te = jnp.sum((trow >= cpc).astype(jnp.int32), axis=1, keepdims=True)
    te_ref[...] = jnp.minimum(te, _E - 1)


def _sc_scatter_x(x_f32, pos_row):
    mesh = plsc.VectorSubcoreMesh(core_axis_name="c", subcore_axis_name="s")
    half = _BPW // 2

    @pl.kernel(
        mesh=mesh,
        out_type=jax.ShapeDtypeStruct((_NT_PAD, _D), jnp.float32),
        scratch_types=[
            pltpu.VMEM((half,), jnp.int32),
            pltpu.VMEM((half, _D), jnp.float32),
            pltpu.SemaphoreType.DMA,
        ],
    )
    def k(x_hbm, idx_hbm, out_hbm, idx_v, rows_v, sem):
        wid = lax.axis_index("s") * _NC + lax.axis_index("c")
        base = wid * _BPW
        src0 = lax.rem(base, _T)
        for r in range(2):
            pltpu.sync_copy(idx_hbm.at[pl.ds(base + r * half, half)], idx_v)
            pltpu.sync_copy(x_hbm.at[pl.ds(src0 + r * half, half)], rows_v)
            pltpu.async_copy(rows_v, out_hbm.at[idx_v], sem).wait()

    return k(x_f32, pos_row)


def _sc_gather_os(os, pos_row):
    mesh = plsc.VectorSubcoreMesh(core_axis_name="c", subcore_axis_name="s")
    half = _BPW // 2

    @pl.kernel(
        mesh=mesh,
        out_type=jax.ShapeDtypeStruct((_P, _D), jnp.float32),
        scratch_types=[
            pltpu.VMEM((half,), jnp.int32),
            pltpu.VMEM((half, _D), jnp.float32),
            pltpu.SemaphoreType.DMA,
        ],
    )
    def k(os_hbm, idx_hbm, out_hbm, idx_v, rows_v, sem):
        wid = lax.axis_index("s") * _NC + lax.axis_index("c")
        base = wid * _BPW
        for r in range(2):
            pltpu.sync_copy(idx_hbm.at[pl.ds(base + r * half, half)], idx_v)
            pltpu.async_copy(os_hbm.at[idx_v], rows_v, sem).wait()
            pltpu.sync_copy(rows_v, out_hbm.at[pl.ds(base + r * half, half)])

    return k(os, pos_row)


def _expert_kernel(te_ref, xs_ref, w1_ref, w2_ref, out_ref):
    del te_ref
    xg = xs_ref[...].astype(jnp.bfloat16)
    h = jnp.dot(xg, w1_ref[0].astype(jnp.bfloat16),
                preferred_element_type=jnp.float32)
    g = h[:, :_FF]
    u = h[:, _FF:]
    a = (jax.nn.silu(g) * u).astype(jnp.bfloat16)
    out_ref[...] = jnp.dot(a, w2_ref[0].astype(jnp.bfloat16),
                           preferred_element_type=jnp.float32)


def _shared_kernel(x_ref, sgu_ref, sd_ref, out_ref):
    xb = x_ref[...].astype(jnp.bfloat16)
    h = jnp.dot(xb, sgu_ref[...], preferred_element_type=jnp.float32)
    g = h[:, :_SFF]
    u = h[:, _SFF:]
    a = (jax.nn.silu(g) * u).astype(jnp.bfloat16)
    out_ref[...] = jnp.dot(a, sd_ref[...], preferred_element_type=jnp.float32)


def _final_kernel(sh_ref, g0_ref, g1_ref, w_ref, out_ref):
    w0 = w_ref[:, 0:1]
    w1 = w_ref[:, 1:2]
    out_ref[...] = sh_ref[...] + w0 * g0_ref[...] + w1 * g1_ref[...]


def kernel(hidden_states, gate_w, w_gate_up, w_down, shared_gate_up, shared_down):
    sgu_bf = shared_gate_up.astype(jnp.bfloat16)
    sd_bf = shared_down.astype(jnp.bfloat16)

    topk_w, pos2, te = pl.pallas_call(
        _router_kernel,
        out_shape=[
            jax.ShapeDtypeStruct((_T, _K), jnp.float32),
            jax.ShapeDtypeStruct((_T, _K), jnp.int32),
            jax.ShapeDtypeStruct((_NUM_TILES, 1), jnp.int32),
        ],
    )(hidden_states, gate_w)

    te_flat = te.reshape(_NUM_TILES)
    pos_row = pos2.T.reshape(_P)

    x_sorted = _sc_scatter_x(hidden_states, pos_row)

    os = pl.pallas_call(
        _expert_kernel,
        grid_spec=pltpu.PrefetchScalarGridSpec(
            num_scalar_prefetch=1,
            grid=(_NUM_TILES,),
            in_specs=[
                pl.BlockSpec((_TILE, _D), lambda t, te: (t, 0)),
                pl.BlockSpec((1, _D, 2 * _FF), lambda t, te: (te[t], 0, 0)),
                pl.BlockSpec((1, _FF, _D), lambda t, te: (te[t], 0, 0)),
            ],
            out_specs=pl.BlockSpec((_TILE, _D), lambda t, te: (t, 0)),
        ),
        out_shape=jax.ShapeDtypeStruct((_NT_PAD, _D), jnp.float32),
    )(te_flat, x_sorted, w_gate_up, w_down)

    shared = pl.pallas_call(
        _shared_kernel,
        grid=(_T // _TT,),
        in_specs=[
            pl.BlockSpec((_TT, _D), lambda i: (i, 0)),
            pl.BlockSpec((_D, 2 * _SFF), lambda i: (0, 0)),
            pl.BlockSpec((_SFF, _D), lambda i: (0, 0)),
        ],
        out_specs=pl.BlockSpec((_TT, _D), lambda i: (i, 0)),
        out_shape=jax.ShapeDtypeStruct((_T, _D), jnp.float32),
    )(hidden_states, sgu_bf, sd_bf)

    g = _sc_gather_os(os, pos_row)

    out = pl.pallas_call(
        _final_kernel,
        grid=(_T // _TT,),
        in_specs=[
            pl.BlockSpec((_TT, _D), lambda i: (i, 0)),
            pl.BlockSpec((_TT, _D), lambda i: (i, 0)),
            pl.BlockSpec((_TT, _D), lambda i: (i + _T // _TT, 0)),
            pl.BlockSpec((_TT, _K), lambda i: (i, 0)),
        ],
        out_specs=pl.BlockSpec((_TT, _D), lambda i: (i, 0)),
        out_shape=jax.ShapeDtypeStruct((_T, _D), jnp.float32),
    )(shared, g, g, topk_w)

    return out
